# baseline (speedup 1.0000x reference)
import jax
import jax.numpy as jnp
from jax.experimental import pallas as pl
from jax.experimental.pallas import tpu as pltpu
from jax.experimental.pallas import tpu_sc as plsc

_VOCAB = 100000
_EMBED = 64
_BATCH = 1024

_V_TILE = 2048

_SC_CORES = 2
_SC_SUBCORES = 16
_SC_WORKERS = _SC_CORES * _SC_SUBCORES
_B_PER_W = _BATCH // _SC_WORKERS


def _sc_gather_pairs(tbl2, xq):
    mesh = plsc.VectorSubcoreMesh(core_axis_name="c", subcore_axis_name="s")

    @pl.kernel(
        out_type=jax.ShapeDtypeStruct((_BATCH, 2 * _EMBED), tbl2.dtype),
        mesh=mesh,
        scratch_types=[
            pltpu.VMEM((_B_PER_W,), jnp.int32),
            pltpu.VMEM((_B_PER_W, 2 * _EMBED), jnp.float32),
            pltpu.SemaphoreType.DMA,
        ],
    )
    def gather_kernel(tbl_hbm, i_hbm, o_hbm, idx_v, rows_v, sem):
        wid = jax.lax.axis_index("s") * _SC_CORES + jax.lax.axis_index("c")
        base = wid * _B_PER_W
        pltpu.sync_copy(i_hbm.at[pl.ds(base, _B_PER_W)], idx_v)
        pltpu.async_copy(tbl_hbm.at[idx_v], rows_v, sem).wait()
        pltpu.sync_copy(rows_v, o_hbm.at[pl.ds(base, _B_PER_W)])

    return gather_kernel(tbl2, xq)


def _sel_body(par_ref, e2_ref, e_ref, ebf_ref):
    e2 = e2_ref[...]
    lo = jax.lax.slice(e2, (0, 0), (_BATCH, _EMBED))
    hi = jax.lax.slice(e2, (0, _EMBED), (_BATCH, 2 * _EMBED))
    e = jnp.where(par_ref[...] == 1, hi, lo)
    e_ref[...] = e
    ebf_ref[...] = e.astype(jnp.bfloat16)


def _tc_select(parity, e2):
    return pl.pallas_call(
        _sel_body,
        out_shape=[
            jax.ShapeDtypeStruct((_BATCH, _EMBED), jnp.float32),
            jax.ShapeDtypeStruct((_BATCH, _EMBED), jnp.bfloat16),
        ],
    )(parity, e2)


_N_CORES = 2
_STEPS = 25
_TILES = 49
_LAST = _TILES - 1
_TAIL = _VOCAB - _LAST * _V_TILE
_NBUF = 4


def _mm_body(e_ref, w_ref, b_ref, o_hbm, acc_ref, tacc_ref, sems, tsem):
    i = pl.program_id(0)
    j = pl.program_id(1)
    t = i * _STEPS + j
    slot = jax.lax.rem(j, _NBUF)
    col = t * _V_TILE

    def full_copy(s, tt):
        return pltpu.make_async_copy(
            acc_ref.at[s], o_hbm.at[:, pl.ds(tt * _V_TILE, _V_TILE)], sems.at[s]
        )

    def tail_copy():
        return pltpu.make_async_copy(
            tacc_ref,
            o_hbm.at[:, pl.ds(_LAST * _V_TILE, _TAIL)],
            tsem,
        )

    @pl.when(jnp.logical_and(j >= _NBUF, t <= _LAST))
    def _():
        full_copy(slot, t - _NBUF).wait()

    @pl.when(t < _LAST)
    def _():
        acc_ref[slot] = jax.lax.dot_general(
            e_ref[...], w_ref[...],
            (((1,), (1,)), ((), ())),
            preferred_element_type=jnp.float32,
        ) + b_ref[...]
        full_copy(slot, t).start()

    @pl.when(t == _LAST)
    def _():
        full = jax.lax.dot_general(
            e_ref[...], w_ref[...],
            (((1,), (1,)), ((), ())),
            preferred_element_type=jnp.float32,
        ) + b_ref[...]
        tacc_ref[...] = jax.lax.slice(full, (0, 0), (_BATCH, _TAIL))
        tail_copy().start()

    @pl.when(jnp.logical_and(j == _STEPS - 1, i == 0))
    def _():
        for k in range(_NBUF):
            full_copy(k, 0).wait()

    @pl.when(jnp.logical_and(j == _STEPS - 1, i == 1))
    def _():
        for k in range(_NBUF - 1):
            full_copy(k, 0).wait()
        tail_copy().wait()


def _tc_project(e_bf, W_bf, b):
    b2 = b.reshape(1, _VOCAB)
    clamp = lambda t: jnp.minimum(t, _LAST)
    return pl.pallas_call(
        _mm_body,
        grid=(_N_CORES, _STEPS),
        in_specs=[
            pl.BlockSpec((_BATCH, _EMBED), lambda i, j: (0, 0)),
            pl.BlockSpec((_V_TILE, _EMBED),
                         lambda i, j: (clamp(i * _STEPS + j), 0)),
            pl.BlockSpec((1, _V_TILE),
                         lambda i, j: (0, clamp(i * _STEPS + j))),
        ],
        out_specs=pl.BlockSpec(memory_space=pl.ANY),
        out_shape=jax.ShapeDtypeStruct((_BATCH, _VOCAB), jnp.float32),
        scratch_shapes=[
            pltpu.VMEM((_NBUF, _BATCH, _V_TILE), jnp.float32),
            pltpu.VMEM((_BATCH, _TAIL), jnp.float32),
            pltpu.SemaphoreType.DMA((_NBUF,)),
            pltpu.SemaphoreType.DMA,
        ],
        compiler_params=pltpu.CompilerParams(
            dimension_semantics=("parallel", "arbitrary"),
        ),
    )(e_bf, W_bf, b2)


def kernel(x, table, W, b):
    xi = x.astype(jnp.int32)
    tbl2 = table.reshape(_VOCAB // 2, 2 * _EMBED)
    e2 = _sc_gather_pairs(tbl2, xi >> 1)
    W_bf = W.astype(jnp.bfloat16)
    parity = (xi & 1).reshape(_BATCH, 1)
    e, e_bf = _tc_select(parity, e2)
    logits = _tc_project(e_bf, W_bf, b)
    return (logits, e)


# speedup vs baseline: 1.0584x; 1.0584x over previous
import jax
import jax.numpy as jnp
from jax.experimental import pallas as pl
from jax.experimental.pallas import tpu as pltpu
from jax.experimental.pallas import tpu_sc as plsc

_VOCAB = 100000
_EMBED = 64
_BATCH = 1024

_V_TILE = 2048

_SC_CORES = 2
_SC_SUBCORES = 16
_SC_WORKERS = _SC_CORES * _SC_SUBCORES
_B_PER_W = _BATCH // _SC_WORKERS


_B_PER_SCS = _BATCH // _SC_CORES


def _sc_gather(table, x):
    mesh = plsc.ScalarSubcoreMesh(axis_name="core", num_cores=_SC_CORES)

    @pl.kernel(
        out_type=jax.ShapeDtypeStruct((_BATCH, _EMBED), table.dtype),
        mesh=mesh,
        scratch_types=[
            pltpu.SMEM((_B_PER_SCS,), jnp.int32),
            pltpu.SemaphoreType.DMA,
            pltpu.SemaphoreType.DMA,
        ],
    )
    def gather_kernel(tbl_hbm, i_hbm, o_hbm, idx_s, isem, sem):
        c = jax.lax.axis_index("core")
        base = c * _B_PER_SCS
        pltpu.async_copy(i_hbm.at[pl.ds(base, _B_PER_SCS)], idx_s, isem).wait()

        @pl.loop(0, _B_PER_SCS)
        def _(i):
            pltpu.async_copy(tbl_hbm.at[idx_s[i]], o_hbm.at[base + i], sem)

        @pl.loop(0, _B_PER_SCS)
        def _(i):
            pltpu.make_async_copy(tbl_hbm.at[0], o_hbm.at[base + i], sem).wait()

    return gather_kernel(table, x)


_N_CORES = 2
_STEPS = 25
_TILES = 49
_LAST = _TILES - 1
_TAIL = _VOCAB - _LAST * _V_TILE
_NBUF = 4


def _mm_body(e_ref, w_ref, b_ref, o_hbm, acc_ref, tacc_ref, sems, tsem):
    i = pl.program_id(0)
    j = pl.program_id(1)
    t = i * _STEPS + j
    slot = jax.lax.rem(j, _NBUF)
    col = t * _V_TILE

    def full_copy(s, tt):
        return pltpu.make_async_copy(
            acc_ref.at[s], o_hbm.at[:, pl.ds(tt * _V_TILE, _V_TILE)], sems.at[s]
        )

    def tail_copy():
        return pltpu.make_async_copy(
            tacc_ref,
            o_hbm.at[:, pl.ds(_LAST * _V_TILE, _TAIL)],
            tsem,
        )

    @pl.when(jnp.logical_and(j >= _NBUF, t <= _LAST))
    def _():
        full_copy(slot, t - _NBUF).wait()

    e_bf = e_ref[...].astype(jnp.bfloat16)

    @pl.when(t < _LAST)
    def _():
        acc_ref[slot] = jax.lax.dot_general(
            e_bf, w_ref[...],
            (((1,), (1,)), ((), ())),
            preferred_element_type=jnp.float32,
        ) + b_ref[...]
        full_copy(slot, t).start()

    @pl.when(t == _LAST)
    def _():
        full = jax.lax.dot_general(
            e_bf, w_ref[...],
            (((1,), (1,)), ((), ())),
            preferred_element_type=jnp.float32,
        ) + b_ref[...]
        tacc_ref[...] = jax.lax.slice(full, (0, 0), (_BATCH, _TAIL))
        tail_copy().start()

    @pl.when(jnp.logical_and(j == _STEPS - 1, i == 0))
    def _():
        for k in range(_NBUF):
            full_copy(k, 0).wait()

    @pl.when(jnp.logical_and(j == _STEPS - 1, i == 1))
    def _():
        for k in range(_NBUF - 1):
            full_copy(k, 0).wait()
        tail_copy().wait()


def _tc_project(e, W_bf, b):
    b2 = b.reshape(1, _VOCAB)
    clamp = lambda t: jnp.minimum(t, _LAST)
    return pl.pallas_call(
        _mm_body,
        grid=(_N_CORES, _STEPS),
        in_specs=[
            pl.BlockSpec((_BATCH, _EMBED), lambda i, j: (0, 0)),
            pl.BlockSpec((_V_TILE, _EMBED),
                         lambda i, j: (clamp(i * _STEPS + j), 0)),
            pl.BlockSpec((1, _V_TILE),
                         lambda i, j: (0, clamp(i * _STEPS + j))),
        ],
        out_specs=pl.BlockSpec(memory_space=pl.ANY),
        out_shape=jax.ShapeDtypeStruct((_BATCH, _VOCAB), jnp.float32),
        scratch_shapes=[
            pltpu.VMEM((_NBUF, _BATCH, _V_TILE), jnp.float32),
            pltpu.VMEM((_BATCH, _TAIL), jnp.float32),
            pltpu.SemaphoreType.DMA((_NBUF,)),
            pltpu.SemaphoreType.DMA,
        ],
        compiler_params=pltpu.CompilerParams(
            dimension_semantics=("parallel", "arbitrary"),
        ),
    )(e, W_bf, b2)


def kernel(x, table, W, b):
    xi = x.astype(jnp.int32)
    e = _sc_gather(table, xi)
    W_bf = W.astype(jnp.bfloat16)
    logits = _tc_project(e, W_bf, b)
    return (logits, e)
